# Initial kernel scaffold; baseline (speedup 1.0000x reference)
import jax
import jax.numpy as jnp
from jax.experimental import pallas as pl


def kernel(x, edge_index, edge_weight, W_mean, b_mean, W_logstd, b_logstd, noise):
    raise NotImplementedError("write your pallas kernel here")


# speedup vs baseline: 5.8608x; 5.8608x over previous
import functools

import jax
import jax.numpy as jnp
from jax import lax
from jax.experimental import pallas as pl
from jax.experimental.pallas import tpu as pltpu
from jax.experimental.pallas import tpu_sc as plsc

N = 10000
E = 320000
F_IN = 128
H = 32
HC = 2 * H

NC = 2
NS = 16
NW = NC * NS

EPT = E // NW
CH = 80
NITER = EPT // CH
RPT = 624
RTAIL = N - NS * RPT


def _mm_body(x_ref, w_ref, o_ref):
    o_ref[...] = jnp.dot(x_ref[...], w_ref[...],
                         preferred_element_type=jnp.float32)


def _support_matmul(x, w_cat):
    bm = 400
    return pl.pallas_call(
        _mm_body,
        grid=(N // bm,),
        in_specs=[
            pl.BlockSpec((bm, F_IN), lambda i: (i, 0)),
            pl.BlockSpec((F_IN, HC), lambda i: (0, 0)),
        ],
        out_specs=pl.BlockSpec((bm, HC), lambda i: (i, 0)),
        out_shape=jax.ShapeDtypeStruct((N, HC), jnp.float32),
    )(x, w_cat)


def _sc_body(support_hbm, src_hbm, dst_hbm, w_hbm, out_hbm,
             acc_sh, src_v, dst_v, w_v, rows_v, sem):
    cid = lax.axis_index("c")
    sid = lax.axis_index("s")

    def _zero_body(i, _):
        for j in range(HC // 16):
            rows_v[i, pl.ds(j * 16, 16)] = jnp.zeros((16,), jnp.float32)
        return 0
    lax.fori_loop(0, CH, _zero_body, 0)
    rbase = pl.multiple_of(sid * RPT, 8)
    for k in range(RPT // CH):
        pltpu.sync_copy(rows_v, acc_sh.at[pl.ds(rbase + k * CH, CH)])
    rem = RPT % CH
    if rem:
        pltpu.sync_copy(rows_v.at[pl.ds(0, rem)],
                        acc_sh.at[pl.ds(rbase + (RPT // CH) * CH, rem)])

    @pl.when(sid == 0)
    def _zero_tail():
        pltpu.sync_copy(rows_v.at[pl.ds(0, RTAIL)],
                        acc_sh.at[pl.ds(NS * RPT, RTAIL)])

    plsc.subcore_barrier()

    ebase = (cid * NS + sid) * EPT

    def _edge_body(it, _):
        off = ebase + it * CH
        pltpu.sync_copy(src_hbm.at[pl.ds(off, CH)], src_v)
        pltpu.sync_copy(dst_hbm.at[pl.ds(off, CH)], dst_v)
        pltpu.sync_copy(w_hbm.at[pl.ds(off, CH)], w_v)
        pltpu.async_copy(support_hbm.at[src_v], rows_v, sem).wait()

        def _scale_body(c, _):
            w16 = w_v[pl.ds(c * 16, 16)]
            for i in range(16):
                r = c * 16 + i
                s = jnp.full((16,), w16[i], jnp.float32)
                for j in range(HC // 16):
                    rows_v[r, pl.ds(j * 16, 16)] = (
                        rows_v[r, pl.ds(j * 16, 16)] * s)
            return 0
        lax.fori_loop(0, CH // 16, _scale_body, 0)

        pltpu.sync_copy(rows_v, acc_sh.at[dst_v], add=True)
        return 0

    lax.fori_loop(0, NITER, _edge_body, 0)
    plsc.subcore_barrier()

    pltpu.sync_copy(acc_sh.at[pl.ds(rbase, RPT)],
                    out_hbm.at[cid, pl.ds(rbase, RPT)])

    @pl.when(sid == 0)
    def _out_tail():
        pltpu.sync_copy(acc_sh.at[pl.ds(NS * RPT, RTAIL)],
                        out_hbm.at[cid, pl.ds(NS * RPT, RTAIL)])


def _sc_spmm(support, src, dst, w):
    mesh = plsc.VectorSubcoreMesh(core_axis_name="c", subcore_axis_name="s",
                                  num_cores=NC, num_subcores=NS)
    f = functools.partial(
        pl.kernel,
        out_type=jax.ShapeDtypeStruct((NC, N, HC), jnp.float32),
        mesh=mesh,
        scratch_types=[
            pltpu.VMEM_SHARED((N, HC), jnp.float32),
            pltpu.VMEM((CH,), jnp.int32),
            pltpu.VMEM((CH,), jnp.int32),
            pltpu.VMEM((CH,), jnp.float32),
            pltpu.VMEM((CH, HC), jnp.float32),
            pltpu.SemaphoreType.DMA,
        ],
        compiler_params=pltpu.CompilerParams(use_tc_tiling_on_sc=False),
    )(_sc_body)
    return f(support, src, dst, w)


def _fin_body(p0m_ref, p0l_ref, p1m_ref, p1l_ref, nz_ref, bm_ref, bl_ref,
              o_ref):
    mean = p0m_ref[...] + p1m_ref[...] + bm_ref[...]
    logstd = p0l_ref[...] + p1l_ref[...] + bl_ref[...]
    o_ref[...] = nz_ref[...] * jnp.exp(logstd) + mean


def _finalize(p0m, p0l, p1m, p1l, noise, b_mean, b_logstd):
    bm = 400
    bspec = pl.BlockSpec((bm, H), lambda i: (i, 0))
    cspec = pl.BlockSpec((1, H), lambda i: (0, 0))
    return pl.pallas_call(
        _fin_body,
        grid=(N // bm,),
        in_specs=[bspec, bspec, bspec, bspec, bspec, cspec, cspec],
        out_specs=bspec,
        out_shape=jax.ShapeDtypeStruct((N, H), jnp.float32),
    )(p0m, p0l, p1m, p1l, noise, b_mean.reshape(1, H),
      b_logstd.reshape(1, H))


def kernel(x, edge_index, edge_weight, W_mean, b_mean, W_logstd, b_logstd,
           noise):
    w_cat = jnp.concatenate([W_mean, W_logstd], axis=1)
    support = _support_matmul(x, w_cat)
    src = edge_index[0].astype(jnp.int32)
    dst = edge_index[1].astype(jnp.int32)
    partials = _sc_spmm(support, src, dst, edge_weight)
    return _finalize(partials[0, :, :H], partials[0, :, H:],
                     partials[1, :, :H], partials[1, :, H:],
                     noise, b_mean, b_logstd)
